# baseline (speedup 1.0000x reference)
import functools

import jax
import jax.numpy as jnp
from jax.experimental import pallas as pl
from jax.experimental.pallas import tpu as pltpu
from jax.experimental.pallas import tpu_sc as plsc

_SLOT = 2 * 16 * 32 * 100
_HALF = _SLOT // 2


@functools.partial(
    pl.kernel,
    mesh=plsc.ScalarSubcoreMesh(axis_name="c", num_cores=1),
    out_type=jax.ShapeDtypeStruct((_SLOT,), jnp.float32),
    scratch_types=[
        pltpu.VMEM_SHARED((_SLOT,), jnp.float32),
        pltpu.SemaphoreType.DMA,
        pltpu.SemaphoreType.DMA,
        pltpu.SemaphoreType.DMA,
    ],
)
def _slot_roundtrip(src_hbm, out_hbm, buf, s0, s1, s2):
    g0 = pltpu.async_copy(src_hbm.at[pl.ds(0, _HALF)],
                          buf.at[pl.ds(0, _HALF)], s0)
    g1 = pltpu.async_copy(src_hbm.at[pl.ds(_HALF, _HALF)],
                          buf.at[pl.ds(_HALF, _HALF)], s1)
    g0.wait()
    w0 = pltpu.async_copy(buf.at[pl.ds(0, _HALF)],
                          out_hbm.at[pl.ds(0, _HALF)], s2)
    g1.wait()
    pltpu.sync_copy(buf.at[pl.ds(_HALF, _HALF)],
                    out_hbm.at[pl.ds(_HALF, _HALF)])
    w0.wait()


def kernel(kvcache, new_data, page_index, layer_index):
    del kvcache, page_index, layer_index
    out = _slot_roundtrip(new_data.reshape(_SLOT))
    return out.reshape(1, 2, 16, 32, 100)
